# baseline (speedup 1.0000x reference)
import functools

import jax
import jax.numpy as jnp
from jax import lax
from jax.experimental import pallas as pl
from jax.experimental.pallas import tpu as pltpu
from jax.experimental.pallas import tpu_sc as plsc


def _sc_info():
    info = plsc.get_sparse_core_info()
    return info.num_cores, info.num_subcores


def _make_sc_linearize(L, B):
    nc, ns = _sc_info()
    nw = nc * ns
    cols_per_w = B // nw
    assert B % nw == 0

    mesh = plsc.VectorSubcoreMesh(core_axis_name="c", subcore_axis_name="s")

    @functools.partial(
        pl.kernel,
        mesh=mesh,
        out_type=jax.ShapeDtypeStruct((B * L,), jnp.int32),
        scratch_types=[pltpu.VMEM((L, cols_per_w), jnp.int32)],
    )
    def linearize_kernel(ex_hbm, out_hbm, idx_v):
        wid = lax.axis_index("s") * nc + lax.axis_index("c")
        c0 = wid * cols_per_w
        pltpu.sync_copy(ex_hbm.at[:, pl.ds(c0, cols_per_w)], idx_v)
        for t in range(L):
            pltpu.sync_copy(idx_v.at[t], out_hbm.at[pl.ds(t * B + c0, cols_per_w)])

    return linearize_kernel


def _make_sc_gather(n_rows, d):
    nc, ns = _sc_info()
    nw = nc * ns
    rows_per_w = n_rows // nw
    chunk = min(rows_per_w, 2048)
    n_chunks = rows_per_w // chunk
    assert rows_per_w % chunk == 0 and n_rows % nw == 0

    mesh = plsc.VectorSubcoreMesh(core_axis_name="c", subcore_axis_name="s")

    @functools.partial(
        pl.kernel,
        mesh=mesh,
        out_type=jax.ShapeDtypeStruct((n_rows, d), jnp.float32),
        scratch_types=[
            pltpu.VMEM((chunk,), jnp.int32),
            pltpu.VMEM((chunk, d), jnp.float32),
            pltpu.SemaphoreType.DMA,
        ],
        compiler_params=pltpu.CompilerParams(use_tc_tiling_on_sc=False),
    )
    def gather_kernel(idx_hbm, table_hbm, out_hbm, idx_v, rows_v, sem):
        wid = lax.axis_index("s") * nc + lax.axis_index("c")
        base = wid * rows_per_w
        for j in range(n_chunks):
            off = base + j * chunk
            pltpu.sync_copy(idx_hbm.at[pl.ds(off, chunk)], idx_v)
            pltpu.async_copy(table_hbm.at[idx_v], rows_v, sem).wait()
            pltpu.sync_copy(rows_v, out_hbm.at[pl.ds(off, chunk)])

    return gather_kernel


def _lstm_body(L, D, bt, x_ref, wih_ref, whh_ref, b_ref, out_ref):
    wih = wih_ref[...]
    whh = whh_ref[...]
    b = b_ref[...]
    hT = jnp.zeros((D, bt), dtype=jnp.float32)
    c = jnp.zeros((D, bt), dtype=jnp.float32)
    cdims = (((1,), (1,)), ((), ()))
    for t in range(L):
        x_t = x_ref[t]
        gT = (
            lax.dot_general(wih, x_t, cdims, preferred_element_type=jnp.float32)
            + jnp.dot(whh, hT, preferred_element_type=jnp.float32)
            + b
        )
        s_if = jax.nn.sigmoid(gT[0:2 * D, :])
        g = jnp.tanh(gT[2 * D:3 * D, :])
        o = jax.nn.sigmoid(gT[3 * D:4 * D, :])
        c = s_if[D:2 * D, :] * c + s_if[0:D, :] * g
        hT = o * jnp.tanh(c)
        out_ref[:, t, :] = jnp.swapaxes(hT, 0, 1)


def _make_tc_lstm(B, L, D, bt):
    grid = (B // bt,)
    body = functools.partial(_lstm_body, L, D, bt)
    return pl.pallas_call(
        body,
        grid=grid,
        in_specs=[
            pl.BlockSpec((L, bt, D), lambda i: (0, i, 0)),
            pl.BlockSpec((4 * D, D), lambda i: (0, 0)),
            pl.BlockSpec((4 * D, D), lambda i: (0, 0)),
            pl.BlockSpec((4 * D, 1), lambda i: (0, 0)),
        ],
        out_specs=pl.BlockSpec((bt, L, D), lambda i: (i, 0, 0)),
        out_shape=jax.ShapeDtypeStruct((B, L, D), jnp.float32),
    )


def kernel(examples, user_cas_embedding, W_ih, W_hh, b_ih, b_hh):
    B, L = examples.shape
    N, D = user_cas_embedding.shape
    ex_t = jnp.swapaxes(examples, 0, 1)
    idx = _make_sc_linearize(L, B)(ex_t)
    gathered = _make_sc_gather(B * L, D)(idx, user_cas_embedding)
    x = gathered.reshape(L, B, D)
    b = (b_ih + b_hh).reshape(4 * D, 1)
    return _make_tc_lstm(B, L, D, 1024)(x, W_ih, W_hh, b)


# speedup vs baseline: 1.1757x; 1.1757x over previous
import functools

import jax
import jax.numpy as jnp
from jax import lax
from jax.experimental import pallas as pl
from jax.experimental.pallas import tpu as pltpu
from jax.experimental.pallas import tpu_sc as plsc


def _sc_info():
    info = plsc.get_sparse_core_info()
    return info.num_cores, info.num_subcores


def _make_sc_linearize(L, B):
    nc, ns = _sc_info()
    nw = nc * ns
    cols_per_w = B // nw
    assert B % nw == 0

    mesh = plsc.VectorSubcoreMesh(core_axis_name="c", subcore_axis_name="s")

    @functools.partial(
        pl.kernel,
        mesh=mesh,
        out_type=jax.ShapeDtypeStruct((B * L,), jnp.int32),
        scratch_types=[pltpu.VMEM((L, cols_per_w), jnp.int32)],
    )
    def linearize_kernel(ex_hbm, out_hbm, idx_v):
        wid = lax.axis_index("s") * nc + lax.axis_index("c")
        c0 = wid * cols_per_w
        pltpu.sync_copy(ex_hbm.at[:, pl.ds(c0, cols_per_w)], idx_v)
        for t in range(L):
            pltpu.sync_copy(idx_v.at[t], out_hbm.at[pl.ds(t * B + c0, cols_per_w)])

    return linearize_kernel


def _make_sc_gather(n_rows, d):
    nc, ns = _sc_info()
    nw = nc * ns
    rows_per_w = n_rows // nw
    chunk = min(rows_per_w, 2048)
    n_chunks = rows_per_w // chunk
    assert rows_per_w % chunk == 0 and n_rows % nw == 0

    mesh = plsc.VectorSubcoreMesh(core_axis_name="c", subcore_axis_name="s")

    @functools.partial(
        pl.kernel,
        mesh=mesh,
        out_type=jax.ShapeDtypeStruct((n_rows, d), jnp.float32),
        scratch_types=[
            pltpu.VMEM((chunk,), jnp.int32),
            pltpu.VMEM((chunk, d), jnp.float32),
            pltpu.SemaphoreType.DMA,
        ],
        compiler_params=pltpu.CompilerParams(use_tc_tiling_on_sc=False),
    )
    def gather_kernel(idx_hbm, table_hbm, out_hbm, idx_v, rows_v, sem):
        wid = lax.axis_index("s") * nc + lax.axis_index("c")
        base = wid * rows_per_w
        for j in range(n_chunks):
            off = base + j * chunk
            pltpu.sync_copy(idx_hbm.at[pl.ds(off, chunk)], idx_v)
            pltpu.async_copy(table_hbm.at[idx_v], rows_v, sem).wait()
            pltpu.sync_copy(rows_v, out_hbm.at[pl.ds(off, chunk)])

    return gather_kernel


def _lstm_body(L, D, bt, x_ref, wih_ref, whh_ref, b_ref, out_ref, acc_ref):
    wih = wih_ref[...]
    whh = whh_ref[...]
    b = b_ref[...]
    hT = jnp.zeros((D, bt), dtype=jnp.float32)
    c = jnp.zeros((D, bt), dtype=jnp.float32)
    cdims = (((1,), (1,)), ((), ()))
    for t in range(L):
        x_t = x_ref[t]
        gT = (
            lax.dot_general(wih, x_t, cdims, preferred_element_type=jnp.float32)
            + jnp.dot(whh, hT, preferred_element_type=jnp.float32)
            + b
        )
        s_if = jax.nn.sigmoid(gT[0:2 * D, :])
        g = jnp.tanh(gT[2 * D:3 * D, :])
        o = jax.nn.sigmoid(gT[3 * D:4 * D, :])
        c = s_if[D:2 * D, :] * c + s_if[0:D, :] * g
        hT = o * jnp.tanh(c)
        acc_ref[t * D:(t + 1) * D, :] = hT
    out_ref[...] = acc_ref[...].T


def _make_tc_lstm(B, L, D, bt):
    grid = (B // bt,)
    body = functools.partial(_lstm_body, L, D, bt)
    return pl.pallas_call(
        body,
        grid=grid,
        in_specs=[
            pl.BlockSpec((L, bt, D), lambda i: (0, i, 0)),
            pl.BlockSpec((4 * D, D), lambda i: (0, 0)),
            pl.BlockSpec((4 * D, D), lambda i: (0, 0)),
            pl.BlockSpec((4 * D, 1), lambda i: (0, 0)),
        ],
        out_specs=pl.BlockSpec((bt, L * D), lambda i: (i, 0)),
        out_shape=jax.ShapeDtypeStruct((B, L * D), jnp.float32),
        scratch_shapes=[pltpu.VMEM((L * D, bt), jnp.float32)],
    )


def kernel(examples, user_cas_embedding, W_ih, W_hh, b_ih, b_hh):
    B, L = examples.shape
    N, D = user_cas_embedding.shape
    ex_t = jnp.swapaxes(examples, 0, 1)
    idx = _make_sc_linearize(L, B)(ex_t)
    gathered = _make_sc_gather(B * L, D)(idx, user_cas_embedding)
    x = gathered.reshape(L, B, D)
    b = (b_ih + b_hh).reshape(4 * D, 1)
    out = _make_tc_lstm(B, L, D, 2048)(x, W_ih, W_hh, b)
    return out.reshape(B, L, D)
